# Initial kernel scaffold; baseline (speedup 1.0000x reference)
import functools

import jax
import jax.numpy as jnp
from jax import lax
from jax.experimental import pallas as pl
from jax.experimental.pallas import tpu as pltpu
from jax.experimental.pallas import tpu_sc as plsc

N_MET = 100000
N_RXN = 100000
E_SUB = 1600000
E_ALL = 3200000
MSG_DIM = 16
HID = 32

NC = 2
NS = 16
L = 16

EC1 = 512
CH1 = 100
E_SUB_PAD = NC * NS * CH1 * EC1
ACC_ROWS = N_MET + L
ZROWS = ACC_ROWS // NS
OROWS = N_RXN // NS

EC2 = 512
CH2 = 196
E_ALL_PAD = NC * NS * CH2 * EC2
N_MET_PAD = 100096
CSL = N_MET_PAD // NS

LN10 = 2.302585092994046


def _mesh():
    return plsc.VectorSubcoreMesh(core_axis_name="c", subcore_axis_name="s")


def _substrate_batch(b, met_v, sto_v, conc_v, w1_v, b1_v, w2_v, b2_v, msg_v):
    m16 = met_v[pl.ds(b * L, L)]
    s16 = sto_v[pl.ds(b * L, L)]
    c16 = plsc.load_gather(conc_v, [m16])
    h = []
    for d in range(HID):
        pre = c16 * w1_v[d, 0] + s16 * w1_v[d, 1] + b1_v[d]
        e2 = jnp.exp(pre + pre)
        h.append(1.0 - 2.0 / (e2 + 1.0))
    rows = b * L + lax.iota(jnp.int32, L)
    for m in range(MSG_DIM):
        acc = h[0] * w2_v[m, 0]
        for d in range(1, HID):
            acc = acc + h[d] * w2_v[m, d]
        acc = acc + b2_v[m]
        cols = jnp.full((L,), m, jnp.int32)
        plsc.store_scatter(msg_v, [rows, cols], acc)


def _k1(conc, met, rxn, sto, w1, b1, w2, b2):
    @functools.partial(
        pl.kernel,
        out_type=jax.ShapeDtypeStruct((NC, N_RXN, MSG_DIM), jnp.float32),
        mesh=_mesh(),
        scratch_types=[
            pltpu.VMEM((N_MET,), jnp.float32),
            pltpu.VMEM((EC1,), jnp.int32),
            pltpu.VMEM((EC1,), jnp.float32),
            pltpu.VMEM((4, 128), jnp.int32),
            pltpu.VMEM((EC1, MSG_DIM), jnp.float32),
            pltpu.VMEM((HID, 2), jnp.float32),
            pltpu.VMEM((HID,), jnp.float32),
            pltpu.VMEM((MSG_DIM, HID), jnp.float32),
            pltpu.VMEM((MSG_DIM,), jnp.float32),
            pltpu.VMEM_SHARED((ACC_ROWS, MSG_DIM), jnp.float32),
        ],
    )
    def k1(conc_h, met_h, rxn_h, sto_h, w1_h, b1_h, w2_h, b2_h, out_h,
           conc_v, met_v, sto_v, rxn_v, msg_v, w1_v, b1_v, w2_v, b2_v, acc_s):
        core = lax.axis_index("c")
        sub = lax.axis_index("s")

        pltpu.sync_copy(conc_h, conc_v)
        pltpu.sync_copy(w1_h, w1_v)
        pltpu.sync_copy(b1_h, b1_v)
        pltpu.sync_copy(w2_h, w2_v)
        pltpu.sync_copy(b2_h, b2_v)

        def zrow(i, c):
            msg_v[i, :] = jnp.zeros((L,), jnp.float32)
            return c
        lax.fori_loop(0, EC1, zrow, 0)

        row0 = sub * ZROWS
        nfull = ZROWS // EC1
        def zacc(i, c):
            pltpu.sync_copy(msg_v, acc_s.at[pl.ds(row0 + i * EC1, EC1)])
            return c
        lax.fori_loop(0, nfull, zacc, 0)
        rem = ZROWS - nfull * EC1
        if rem:
            pltpu.sync_copy(msg_v.at[pl.ds(0, rem)],
                            acc_s.at[pl.ds(row0 + nfull * EC1, rem)])
        plsc.subcore_barrier()

        ebase = core * (NS * CH1 * EC1) + sub * (CH1 * EC1)

        def chunk(ci, c):
            eb = ebase + ci * EC1
            pltpu.sync_copy(met_h.at[pl.ds(eb, EC1)], met_v)
            pltpu.sync_copy(sto_h.at[pl.ds(eb, EC1)], sto_v)
            for j in range(4):
                pltpu.sync_copy(rxn_h.at[pl.ds(eb + j * 128, 128)],
                                rxn_v.at[j])

            def batch(b, cc):
                _substrate_batch(b, met_v, sto_v, conc_v, w1_v, b1_v, w2_v,
                                 b2_v, msg_v)
                return cc
            lax.fori_loop(0, EC1 // L, batch, 0)

            for j in range(4):
                pltpu.sync_copy(msg_v.at[pl.ds(j * 128, 128)],
                                acc_s.at[rxn_v.at[j]], add=True)
            return c
        lax.fori_loop(0, CH1, chunk, 0)

        plsc.subcore_barrier()
        orow0 = sub * OROWS
        pltpu.sync_copy(acc_s.at[pl.ds(orow0, OROWS)],
                        out_h.at[core].at[pl.ds(orow0, OROWS)])

    return k1(conc, met, rxn, sto, w1, b1, w2, b2)


def _k2_body(hacc_ref, v1_ref, c1_ref, v2_ref, c2_ref, lk_ref, v_ref):
    h = hacc_ref[0] + hacc_ref[1]
    g = jnp.tanh(
        jax.lax.dot_general(h, v1_ref[...],
                            (((1,), (1,)), ((), ())),
                            preferred_element_type=jnp.float32)
        + c1_ref[...])
    r = jax.lax.dot_general(g, v2_ref[...],
                            (((1,), (1,)), ((), ())),
                            preferred_element_type=jnp.float32) + c2_ref[...]
    v_ref[...] = jnp.exp(lk_ref[...] * LN10) * r


def _k2(hacc, v1, c1, v2, c2, log_k):
    blk = 10000
    grid = N_RXN // blk
    return pl.pallas_call(
        _k2_body,
        grid=(grid,),
        in_specs=[
            pl.BlockSpec((2, blk, MSG_DIM), lambda i: (0, i, 0)),
            pl.BlockSpec((HID, MSG_DIM), lambda i: (0, 0)),
            pl.BlockSpec((1, HID), lambda i: (0, 0)),
            pl.BlockSpec((1, HID), lambda i: (0, 0)),
            pl.BlockSpec((1, 1), lambda i: (0, 0)),
            pl.BlockSpec((blk, 1), lambda i: (i, 0)),
        ],
        out_specs=pl.BlockSpec((blk, 1), lambda i: (i, 0)),
        out_shape=jax.ShapeDtypeStruct((N_RXN, 1), jnp.float32),
    )(hacc, v1, c1, v2, c2, log_k)


def _k34(v, rxn, sto, met):
    @functools.partial(
        pl.kernel,
        out_type=(jax.ShapeDtypeStruct((NC, N_MET_PAD), jnp.float32),
                  jax.ShapeDtypeStruct((E_ALL_PAD,), jnp.float32)),
        mesh=_mesh(),
        scratch_types=[
            pltpu.VMEM((N_MET_PAD,), jnp.float32),
            pltpu.VMEM((EC2,), jnp.int32),
            pltpu.VMEM((EC2,), jnp.float32),
            pltpu.VMEM((EC2,), jnp.float32),
            pltpu.VMEM((CSL,), jnp.float32),
            pltpu.VMEM((CSL,), jnp.float32),
            pltpu.VMEM_SHARED((NS, N_MET_PAD), jnp.float32),
        ],
    )
    def k34(v_h, rxn_h, sto_h, met_h, dacc_h, contrib_h,
            tbl_v, idx_v, sto_v, con_v, rin_v, racc_v, red_s):
        core = lax.axis_index("c")
        sub = lax.axis_index("s")
        ebase = core * (NS * CH2 * EC2) + sub * (CH2 * EC2)

        pltpu.sync_copy(v_h, tbl_v.at[pl.ds(0, N_MET)])

        def chunk1(ci, c):
            eb = ebase + ci * EC2
            pltpu.sync_copy(rxn_h.at[pl.ds(eb, EC2)], idx_v)
            pltpu.sync_copy(sto_h.at[pl.ds(eb, EC2)], sto_v)

            def batch(b, cc):
                r16 = idx_v[pl.ds(b * L, L)]
                s16 = sto_v[pl.ds(b * L, L)]
                v16 = plsc.load_gather(tbl_v, [r16])
                con_v[pl.ds(b * L, L)] = v16 * s16
                return cc
            lax.fori_loop(0, EC2 // L, batch, 0)
            pltpu.sync_copy(con_v, contrib_h.at[pl.ds(eb, EC2)])
            return c
        lax.fori_loop(0, CH2, chunk1, 0)

        def ztbl(i, c):
            tbl_v[pl.ds(i * L, L)] = jnp.zeros((L,), jnp.float32)
            return c
        lax.fori_loop(0, N_MET_PAD // L, ztbl, 0)

        def chunk2(ci, c):
            eb = ebase + ci * EC2
            pltpu.sync_copy(met_h.at[pl.ds(eb, EC2)], idx_v)
            pltpu.sync_copy(contrib_h.at[pl.ds(eb, EC2)], con_v)

            def batch(b, cc):
                m16 = idx_v[pl.ds(b * L, L)]
                c16 = con_v[pl.ds(b * L, L)]
                plsc.addupdate_scatter(tbl_v, [m16], c16)
                return cc
            lax.fori_loop(0, EC2 // L, batch, 0)
            return c
        lax.fori_loop(0, CH2, chunk2, 0)

        pltpu.sync_copy(tbl_v, red_s.at[sub])
        plsc.subcore_barrier()
        col0 = sub * CSL
        pltpu.sync_copy(red_s.at[0].at[pl.ds(col0, CSL)], racc_v)

        def redj(j, c):
            pltpu.sync_copy(red_s.at[j].at[pl.ds(col0, CSL)], rin_v)

            def addi(i, cc):
                racc_v[pl.ds(i * L, L)] = (racc_v[pl.ds(i * L, L)]
                                           + rin_v[pl.ds(i * L, L)])
                return cc
            lax.fori_loop(0, CSL // L, addi, 0)
            return c
        lax.fori_loop(1, NS, redj, 0)
        pltpu.sync_copy(racc_v, dacc_h.at[core].at[pl.ds(col0, CSL)])

    return k34(v, rxn, sto, met)


def _k5_body(d0_ref, d1_ref, conc_ref, p_ref, out_ref):
    lam = p_ref[0, 0]
    tgt = p_ref[0, 1]
    out_ref[...] = (d0_ref[...] + d1_ref[...]
                    - lam * (conc_ref[...] - tgt))


def _k5(d0, d1, conc, p):
    return pl.pallas_call(
        _k5_body,
        out_shape=jax.ShapeDtypeStruct((N_MET, 1), jnp.float32),
    )(d0, d1, conc, p)


def kernel(x, met_sub, rxn_sub, sto_sub, met_all, rxn_all, sto_all,
           W1, b1, W2, b2, V1, c1, V2, c2, log_k, p):
    conc = x[:, 3]
    met_s = met_sub.astype(jnp.int32)
    rxn_s = rxn_sub.astype(jnp.int32)
    met_a = met_all.astype(jnp.int32)
    rxn_a = rxn_all.astype(jnp.int32)

    ps = E_SUB_PAD - E_SUB
    met_s = jnp.pad(met_s, (0, ps))
    rxn_s = jnp.pad(rxn_s, (0, ps), constant_values=N_RXN)
    sto_s = jnp.pad(sto_sub, (0, ps))

    hacc = _k1(conc, met_s, rxn_s, sto_s, W1, b1, W2, b2)
    v = _k2(hacc, V1, c1[None, :], V2, c2[None, :], log_k[:, None])

    pa = E_ALL_PAD - E_ALL
    met_a = jnp.pad(met_a, (0, pa))
    rxn_a = jnp.pad(rxn_a, (0, pa))
    sto_a = jnp.pad(sto_all, (0, pa))

    dacc, _ = _k34(v[:, 0], rxn_a, sto_a, met_a)
    dxdt = _k5(dacc[0, :N_MET, None], dacc[1, :N_MET, None],
               conc[:, None], p)
    return dxdt


# speedup vs baseline: 19.2946x; 19.2946x over previous
import functools

import jax
import jax.numpy as jnp
from jax import lax
from jax.experimental import pallas as pl
from jax.experimental.pallas import tpu as pltpu
from jax.experimental.pallas import tpu_sc as plsc

N_MET = 100000
N_RXN = 100000
E_SUB = 1600000
E_ALL = 3200000
MSG_DIM = 16
HID = 32

NC = 2
NS = 16
L = 16

EC1 = 512
CH1 = 100
E_SUB_PAD = NC * NS * CH1 * EC1
ACC_ROWS = 100096
ZROWS = ACC_ROWS // NS

EC2 = 512
CH2 = 196
E_ALL_PAD = NC * NS * CH2 * EC2
N_MET_PAD = 100096
NW = NC * NS

LN10 = 2.302585092994046

_SC_PARAMS = dict(
    compiler_params=pltpu.CompilerParams(
        needs_layout_passes=False, use_tc_tiling_on_sc=False),
)


def _mesh():
    return plsc.VectorSubcoreMesh(core_axis_name="c", subcore_axis_name="s")


def _substrate_group(g, crows_v, sto_v, w1_v, b1_v, w2_v, b2_v, msg_v):
    cA = crows_v[pl.ds(g * 2 * L, L)]
    cB = crows_v[pl.ds(g * 2 * L + L, L)]
    sA = sto_v[pl.ds(g * 2 * L, L)]
    sB = sto_v[pl.ds(g * 2 * L + L, L)]
    msgA = [b2_v[m, :] for m in range(MSG_DIM)]
    msgB = [b2_v[m, :] for m in range(MSG_DIM)]
    for d in range(HID):
        w0 = w1_v[d, 0, :]
        w1 = w1_v[d, 1, :]
        bb = b1_v[d, :]
        preA = cA * w0 + sA * w1 + bb
        preB = cB * w0 + sB * w1 + bb
        hA = 1.0 - 2.0 / (jnp.exp(preA + preA) + 1.0)
        hB = 1.0 - 2.0 / (jnp.exp(preB + preB) + 1.0)
        for m in range(MSG_DIM):
            w = w2_v[d, m, :]
            msgA[m] = msgA[m] + hA * w
            msgB[m] = msgB[m] + hB * w
    rowsA = g * 2 * L + lax.iota(jnp.int32, L)
    rowsB = rowsA + L
    for m in range(MSG_DIM):
        cols = jnp.full((L,), m, jnp.int32)
        plsc.store_scatter(msg_v, [rowsA, cols], msgA[m])
        plsc.store_scatter(msg_v, [rowsB, cols], msgB[m])


def _k1(conc, met, rxn, sto, w1, b1, w2, b2):
    @functools.partial(
        pl.kernel,
        out_type=jax.ShapeDtypeStruct((NC, ACC_ROWS, MSG_DIM), jnp.float32),
        mesh=_mesh(),
        scratch_types=[
            pltpu.VMEM((4, 128), jnp.int32),
            pltpu.VMEM((EC1,), jnp.float32),
            pltpu.VMEM((4, 128), jnp.int32),
            pltpu.VMEM((EC1,), jnp.float32),
            pltpu.VMEM((EC1, MSG_DIM), jnp.float32),
            pltpu.VMEM((HID, 2, L), jnp.float32),
            pltpu.VMEM((HID, L), jnp.float32),
            pltpu.VMEM((HID, MSG_DIM, L), jnp.float32),
            pltpu.VMEM((MSG_DIM, L), jnp.float32),
            pltpu.VMEM_SHARED((ACC_ROWS, MSG_DIM), jnp.float32),
            pltpu.SemaphoreType.DMA,
        ],
        **_SC_PARAMS,
    )
    def k1(conc_h, met_h, rxn_h, sto_h, w1_h, b1_h, w2_h, b2_h, out_h,
           met_v, sto_v, rxn_v, crows_v, msg_v, w1_v, b1_v, w2_v, b2_v,
           acc_s, sem):
        core = lax.axis_index("c")
        sub = lax.axis_index("s")

        pltpu.sync_copy(w1_h, w1_v)
        pltpu.sync_copy(b1_h, b1_v)
        pltpu.sync_copy(w2_h, w2_v)
        pltpu.sync_copy(b2_h, b2_v)

        def zrow(i, c):
            msg_v[i, :] = jnp.zeros((L,), jnp.float32)
            return c
        lax.fori_loop(0, EC1, zrow, 0)

        row0 = sub * ZROWS
        nfull = ZROWS // EC1
        def zacc(i, c):
            pltpu.sync_copy(msg_v, acc_s.at[pl.ds(row0 + i * EC1, EC1)])
            return c
        lax.fori_loop(0, nfull, zacc, 0)
        rem = ZROWS - nfull * EC1
        if rem:
            pltpu.sync_copy(msg_v.at[pl.ds(0, rem)],
                            acc_s.at[pl.ds(row0 + nfull * EC1, rem)])
        plsc.subcore_barrier()

        ebase = core * (NS * CH1 * EC1) + sub * (CH1 * EC1)

        def chunk(ci, c):
            eb = ebase + ci * EC1
            pltpu.sync_copy(sto_h.at[pl.ds(eb, EC1)], sto_v)
            rb = eb // 128
            pltpu.sync_copy(met_h.at[pl.ds(rb, 4)], met_v)
            pltpu.sync_copy(rxn_h.at[pl.ds(rb, 4)], rxn_v)
            for j in range(4):
                pltpu.async_copy(conc_h.at[met_v.at[j]],
                                 crows_v.at[pl.ds(j * 128, 128)], sem).wait()

            def group(g, cc):
                _substrate_group(g, crows_v, sto_v, w1_v, b1_v, w2_v,
                                 b2_v, msg_v)
                return cc
            lax.fori_loop(0, EC1 // (2 * L), group, 0)

            for j in range(4):
                pltpu.sync_copy(msg_v.at[pl.ds(j * 128, 128)],
                                acc_s.at[rxn_v.at[j]], add=True)
            return c
        lax.fori_loop(0, CH1, chunk, 0)

        plsc.subcore_barrier()
        pltpu.sync_copy(acc_s.at[pl.ds(row0, ZROWS)],
                        out_h.at[core].at[pl.ds(row0, ZROWS)])

    return k1(conc, met, rxn, sto, w1, b1, w2, b2)


def _k2_body(hacc_ref, v1_ref, c1_ref, v2_ref, c2_ref, lk_ref, v_ref):
    h = hacc_ref[0] + hacc_ref[1]
    g = jnp.tanh(
        jax.lax.dot_general(h, v1_ref[...],
                            (((1,), (1,)), ((), ())),
                            preferred_element_type=jnp.float32)
        + c1_ref[...])
    r = jax.lax.dot_general(g, v2_ref[...],
                            (((1,), (0,)), ((), ())),
                            preferred_element_type=jnp.float32) + c2_ref[0]
    v_ref[...] = jnp.exp(lk_ref[...] * LN10) * r


def _k2(hacc, v1, c1, v2, c2, log_k):
    blk = 10000
    grid = N_RXN // blk
    return pl.pallas_call(
        _k2_body,
        grid=(grid,),
        in_specs=[
            pl.BlockSpec((2, blk, MSG_DIM), lambda i: (0, i, 0)),
            pl.BlockSpec((HID, MSG_DIM), lambda i: (0, 0)),
            pl.BlockSpec((1, HID), lambda i: (0, 0)),
            pl.BlockSpec((HID, 1), lambda i: (0, 0)),
            pl.BlockSpec(memory_space=pltpu.SMEM),
            pl.BlockSpec((blk, 1), lambda i: (i, 0)),
        ],
        out_specs=pl.BlockSpec((blk, 1), lambda i: (i, 0)),
        out_shape=jax.ShapeDtypeStruct((N_RXN, 1), jnp.float32),
    )(hacc, v1, c1, v2, c2, log_k)


def _k34(v, rxn, sto, met):
    @functools.partial(
        pl.kernel,
        out_type=(jax.ShapeDtypeStruct((NW, N_MET_PAD), jnp.float32),
                  jax.ShapeDtypeStruct((E_ALL_PAD,), jnp.float32)),
        mesh=_mesh(),
        scratch_types=[
            pltpu.VMEM((N_MET_PAD,), jnp.float32),
            pltpu.VMEM((EC2,), jnp.int32),
            pltpu.VMEM((EC2,), jnp.float32),
            pltpu.VMEM((EC2,), jnp.float32),
        ],
        **_SC_PARAMS,
    )
    def k34(v_h, rxn_h, sto_h, met_h, dacc_h, contrib_h,
            tbl_v, idx_v, sto_v, con_v):
        core = lax.axis_index("c")
        sub = lax.axis_index("s")
        wid = core * NS + sub
        ebase = core * (NS * CH2 * EC2) + sub * (CH2 * EC2)

        pltpu.sync_copy(v_h, tbl_v.at[pl.ds(0, N_MET)])

        def chunk1(ci, c):
            eb = ebase + ci * EC2
            pltpu.sync_copy(rxn_h.at[pl.ds(eb, EC2)], idx_v)
            pltpu.sync_copy(sto_h.at[pl.ds(eb, EC2)], sto_v)

            def batch(b, cc):
                r16 = idx_v[pl.ds(b * L, L)]
                s16 = sto_v[pl.ds(b * L, L)]
                v16 = plsc.load_gather(tbl_v, [r16])
                con_v[pl.ds(b * L, L)] = v16 * s16
                return cc
            lax.fori_loop(0, EC2 // L, batch, 0)
            pltpu.sync_copy(con_v, contrib_h.at[pl.ds(eb, EC2)])
            return c
        lax.fori_loop(0, CH2, chunk1, 0)

        def ztbl(i, c):
            tbl_v[pl.ds(i * L, L)] = jnp.zeros((L,), jnp.float32)
            return c
        lax.fori_loop(0, N_MET_PAD // L, ztbl, 0)

        def chunk2(ci, c):
            eb = ebase + ci * EC2
            pltpu.sync_copy(met_h.at[pl.ds(eb, EC2)], idx_v)
            pltpu.sync_copy(contrib_h.at[pl.ds(eb, EC2)], con_v)

            def batch(b, cc):
                m16 = idx_v[pl.ds(b * L, L)]
                c16 = con_v[pl.ds(b * L, L)]
                plsc.addupdate_scatter(tbl_v, [m16], c16)
                return cc
            lax.fori_loop(0, EC2 // L, batch, 0)
            return c
        lax.fori_loop(0, CH2, chunk2, 0)

        pltpu.sync_copy(tbl_v, dacc_h.at[wid])

    return k34(v, rxn, sto, met)


def _k5_body(dacc_ref, conc_ref, p_ref, out_ref):
    lam = p_ref[0, 0]
    tgt = p_ref[0, 1]
    dsum = jnp.sum(dacc_ref[...], axis=0)[:N_MET]
    out_ref[0, :] = dsum - lam * (conc_ref[0] - tgt)


def _k5(dacc, conc, p):
    return pl.pallas_call(
        _k5_body,
        in_specs=[
            pl.BlockSpec((NW, N_MET_PAD), lambda: (0, 0)),
            pl.BlockSpec((1, N_MET), lambda: (0, 0)),
            pl.BlockSpec(memory_space=pltpu.SMEM),
        ],
        out_specs=pl.BlockSpec((1, N_MET), lambda: (0, 0)),
        out_shape=jax.ShapeDtypeStruct((1, N_MET), jnp.float32),
    )(dacc, conc, p)


def kernel(x, met_sub, rxn_sub, sto_sub, met_all, rxn_all, sto_all,
           W1, b1, W2, b2, V1, c1, V2, c2, log_k, p):
    conc = x[:, 3]
    met_s = met_sub.astype(jnp.int32)
    rxn_s = rxn_sub.astype(jnp.int32)
    met_a = met_all.astype(jnp.int32)
    rxn_a = rxn_all.astype(jnp.int32)

    ps = E_SUB_PAD - E_SUB
    met_s = jnp.pad(met_s, (0, ps))
    rxn_s = jnp.pad(rxn_s, (0, ps), constant_values=N_RXN)
    sto_s = jnp.pad(sto_sub, (0, ps))

    w1s = jnp.broadcast_to(W1[:, :, None], (HID, 2, L)) + 0.0
    b1s = jnp.broadcast_to(b1[:, None], (HID, L)) + 0.0
    w2s = jnp.broadcast_to(W2.T[:, :, None], (HID, MSG_DIM, L)) + 0.0
    b2s = jnp.broadcast_to(b2[:, None], (MSG_DIM, L)) + 0.0

    hacc = _k1(conc, met_s.reshape(-1, 128), rxn_s.reshape(-1, 128),
               sto_s, w1s, b1s, w2s, b2s)
    v = _k2(hacc[:, :N_RXN], V1, c1[None, :], V2.T + 0.0, c2,
            log_k[:, None])

    pa = E_ALL_PAD - E_ALL
    met_a = jnp.pad(met_a, (0, pa))
    rxn_a = jnp.pad(rxn_a, (0, pa))
    sto_a = jnp.pad(sto_all, (0, pa))

    dacc, _ = _k34(v[:, 0], rxn_a, sto_a, met_a)
    dxdt = _k5(dacc, conc[None, :], p)
    return dxdt.reshape(N_MET, 1)


# speedup vs baseline: 22.5510x; 1.1688x over previous
import functools

import jax
import jax.numpy as jnp
from jax import lax
from jax.experimental import pallas as pl
from jax.experimental.pallas import tpu as pltpu
from jax.experimental.pallas import tpu_sc as plsc

N_MET = 100000
N_RXN = 100000
E_SUB = 1600000
E_ALL = 3200000
MSG_DIM = 16
HID = 32

NC = 2
NS = 16
L = 16

EC1 = 384
CH1 = 132
E_SUB_PAD = NC * NS * CH1 * EC1
ACC_ROWS = 100016
ZROWS = ACC_ROWS // NS

EC2 = 512
CH2 = 196
E_ALL_PAD = NC * NS * CH2 * EC2
N_MET_PAD = 100096
NW = NC * NS

LN10 = 2.302585092994046

_SC_PARAMS = dict(
    compiler_params=pltpu.CompilerParams(
        needs_layout_passes=False, use_tc_tiling_on_sc=False),
)


def _mesh():
    return plsc.VectorSubcoreMesh(core_axis_name="c", subcore_axis_name="s")


def _substrate_group(g, crows_v, sto_v, w1_v, b1_v, w2_v, b2_v, msg_v):
    cA = crows_v[pl.ds(g * 2 * L, L)]
    cB = crows_v[pl.ds(g * 2 * L + L, L)]
    sA = sto_v[pl.ds(g * 2 * L, L)]
    sB = sto_v[pl.ds(g * 2 * L + L, L)]
    msgA = [b2_v[m, :] for m in range(MSG_DIM)]
    msgB = [b2_v[m, :] for m in range(MSG_DIM)]
    for d in range(HID):
        w0 = w1_v[d, 0, :]
        w1 = w1_v[d, 1, :]
        bb = b1_v[d, :]
        preA = cA * w0 + sA * w1 + bb
        preB = cB * w0 + sB * w1 + bb
        hA = 1.0 - 2.0 / (jnp.exp(preA + preA) + 1.0)
        hB = 1.0 - 2.0 / (jnp.exp(preB + preB) + 1.0)
        for m in range(MSG_DIM):
            w = w2_v[d, m, :]
            msgA[m] = msgA[m] + hA * w
            msgB[m] = msgB[m] + hB * w
    rowsA = g * 2 * L + lax.iota(jnp.int32, L)
    rowsB = rowsA + L
    for m in range(MSG_DIM):
        cols = jnp.full((L,), m, jnp.int32)
        plsc.store_scatter(msg_v, [rowsA, cols], msgA[m])
        plsc.store_scatter(msg_v, [rowsB, cols], msgB[m])


def _k1(conc, met, rxn, sto, w1, b1, w2, b2):
    @functools.partial(
        pl.kernel,
        out_type=jax.ShapeDtypeStruct((NC, ACC_ROWS, MSG_DIM), jnp.float32),
        mesh=_mesh(),
        scratch_types=[
            pltpu.VMEM((2, 3, 128), jnp.int32),
            pltpu.VMEM((2, EC1), jnp.float32),
            pltpu.VMEM((2, 3, 128), jnp.int32),
            pltpu.VMEM((2, 3, 128), jnp.int32),
            pltpu.VMEM((2, EC1), jnp.float32),
            pltpu.VMEM((2, EC1, MSG_DIM), jnp.float32),
            pltpu.VMEM((HID, 2, L), jnp.float32),
            pltpu.VMEM((HID, L), jnp.float32),
            pltpu.VMEM((HID, MSG_DIM, L), jnp.float32),
            pltpu.VMEM((MSG_DIM, L), jnp.float32),
            pltpu.VMEM_SHARED((ACC_ROWS, MSG_DIM), jnp.float32),
            pltpu.SemaphoreType.DMA,
            pltpu.SemaphoreType.DMA,
            pltpu.SemaphoreType.DMA,
        ],
        **_SC_PARAMS,
    )
    def k1(conc_h, met_h, rxn_h, sto_h, w1_h, b1_h, w2_h, b2_h, out_h,
           met_v, sto_v, rxn_v, rxn_sc, crows_v, msg_v, w1_v, b1_v, w2_v,
           b2_v, acc_s, sem_in, sem_g, sem_sc):
        core = lax.axis_index("c")
        sub = lax.axis_index("s")

        pltpu.sync_copy(w1_h, w1_v)
        pltpu.sync_copy(b1_h, b1_v)
        pltpu.sync_copy(w2_h, w2_v)
        pltpu.sync_copy(b2_h, b2_v)

        def zrow(i, c):
            msg_v[0, i, :] = jnp.zeros((L,), jnp.float32)
            return c
        lax.fori_loop(0, EC1, zrow, 0)

        row0 = sub * ZROWS
        nfull = ZROWS // EC1
        def zacc(i, c):
            pltpu.sync_copy(msg_v.at[0], acc_s.at[pl.ds(row0 + i * EC1, EC1)])
            return c
        lax.fori_loop(0, nfull, zacc, 0)
        rem = ZROWS - nfull * EC1
        if rem:
            pltpu.sync_copy(msg_v.at[0].at[pl.ds(0, rem)],
                            acc_s.at[pl.ds(row0 + nfull * EC1, rem)])
        plsc.subcore_barrier()

        ebase = core * (NS * CH1 * EC1) + sub * (CH1 * EC1)

        def fire_in(ci, b):
            eb = ebase + ci * EC1
            rb = eb // 128
            ds = [pltpu.async_copy(sto_h.at[pl.ds(eb, EC1)], sto_v.at[b],
                                   sem_in),
                  pltpu.async_copy(met_h.at[pl.ds(rb, 3)], met_v.at[b],
                                   sem_in),
                  pltpu.async_copy(rxn_h.at[pl.ds(rb, 3)], rxn_v.at[b],
                                   sem_in)]
            return ds

        def fire_gather(b):
            return [pltpu.async_copy(conc_h.at[met_v.at[b].at[j]],
                                     crows_v.at[b].at[pl.ds(j * 128, 128)],
                                     sem_g)
                    for j in range(3)]

        def drain_gather(b):
            for j in range(3):
                pltpu.make_async_copy(
                    conc_h.at[met_v.at[b].at[j]],
                    crows_v.at[b].at[pl.ds(j * 128, 128)], sem_g).wait()

        def fire_scatter(b):
            for j in range(3):
                for k in range(8):
                    rxn_sc[b, j, pl.ds(k * L, L)] = rxn_v[b, j,
                                                          pl.ds(k * L, L)]
            for j in range(3):
                pltpu.async_copy(msg_v.at[b].at[pl.ds(j * 128, 128)],
                                 acc_s.at[rxn_sc.at[b].at[j]], sem_sc,
                                 add=True)

        def drain_scatter(b):
            for j in range(3):
                pltpu.make_async_copy(msg_v.at[b].at[pl.ds(j * 128, 128)],
                                      acc_s.at[rxn_sc.at[b].at[j]],
                                      sem_sc).wait()

        def compute(b):
            def group(g, cc):
                _substrate_group(g, crows_v.at[b], sto_v.at[b], w1_v, b1_v,
                                 w2_v, b2_v, msg_v.at[b])
                return cc
            lax.fori_loop(0, EC1 // (2 * L), group, 0)

        for d in fire_in(0, 0):
            d.wait()
        fire_gather(0)

        half = CH1 // 2

        def body(i2, c):
            ca = 2 * i2
            cb = ca + 1
            din = fire_in(cb, 1)
            drain_gather(0)

            @pl.when(i2 >= 1)
            def _():
                drain_scatter(0)
            compute(0)
            fire_scatter(0)
            for d in din:
                d.wait()
            fire_gather(1)

            @pl.when(i2 < half - 1)
            def _():
                din2 = fire_in(ca + 2, 0)
                for d in din2:
                    d.wait()
            drain_gather(1)

            @pl.when(i2 >= 1)
            def _():
                drain_scatter(1)
            compute(1)
            fire_scatter(1)

            @pl.when(i2 < half - 1)
            def _():
                fire_gather(0)
            return c
        lax.fori_loop(0, half, body, 0)
        drain_scatter(0)
        drain_scatter(1)

        plsc.subcore_barrier()
        pltpu.sync_copy(acc_s.at[pl.ds(row0, ZROWS)],
                        out_h.at[core].at[pl.ds(row0, ZROWS)])

    return k1(conc, met, rxn, sto, w1, b1, w2, b2)


def _k2_body(hacc_ref, v1_ref, c1_ref, v2_ref, c2_ref, lk_ref, v_ref):
    h = hacc_ref[0] + hacc_ref[1]
    g = jnp.tanh(
        jax.lax.dot_general(h, v1_ref[...],
                            (((1,), (1,)), ((), ())),
                            preferred_element_type=jnp.float32)
        + c1_ref[...])
    r = jax.lax.dot_general(g, v2_ref[...],
                            (((1,), (0,)), ((), ())),
                            preferred_element_type=jnp.float32) + c2_ref[0]
    v_ref[...] = jnp.exp(lk_ref[...] * LN10) * r


def _k2(hacc, v1, c1, v2, c2, log_k):
    blk = 10000
    grid = N_RXN // blk
    return pl.pallas_call(
        _k2_body,
        grid=(grid,),
        in_specs=[
            pl.BlockSpec((2, blk, MSG_DIM), lambda i: (0, i, 0)),
            pl.BlockSpec((HID, MSG_DIM), lambda i: (0, 0)),
            pl.BlockSpec((1, HID), lambda i: (0, 0)),
            pl.BlockSpec((HID, 1), lambda i: (0, 0)),
            pl.BlockSpec(memory_space=pltpu.SMEM),
            pl.BlockSpec((blk, 1), lambda i: (i, 0)),
        ],
        out_specs=pl.BlockSpec((blk, 1), lambda i: (i, 0)),
        out_shape=jax.ShapeDtypeStruct((N_RXN, 1), jnp.float32),
    )(hacc, v1, c1, v2, c2, log_k)


def _k34(v, rxn, sto, met):
    @functools.partial(
        pl.kernel,
        out_type=(jax.ShapeDtypeStruct((NW, N_MET_PAD), jnp.float32),
                  jax.ShapeDtypeStruct((E_ALL_PAD,), jnp.float32)),
        mesh=_mesh(),
        scratch_types=[
            pltpu.VMEM((N_MET_PAD,), jnp.float32),
            pltpu.VMEM((EC2,), jnp.int32),
            pltpu.VMEM((EC2,), jnp.float32),
            pltpu.VMEM((EC2,), jnp.float32),
        ],
        **_SC_PARAMS,
    )
    def k34(v_h, rxn_h, sto_h, met_h, dacc_h, contrib_h,
            tbl_v, idx_v, sto_v, con_v):
        core = lax.axis_index("c")
        sub = lax.axis_index("s")
        wid = core * NS + sub
        ebase = core * (NS * CH2 * EC2) + sub * (CH2 * EC2)

        pltpu.sync_copy(v_h, tbl_v.at[pl.ds(0, N_MET)])

        def chunk1(ci, c):
            eb = ebase + ci * EC2
            pltpu.sync_copy(rxn_h.at[pl.ds(eb, EC2)], idx_v)
            pltpu.sync_copy(sto_h.at[pl.ds(eb, EC2)], sto_v)

            def batch(b, cc):
                r16 = idx_v[pl.ds(b * L, L)]
                s16 = sto_v[pl.ds(b * L, L)]
                v16 = plsc.load_gather(tbl_v, [r16])
                con_v[pl.ds(b * L, L)] = v16 * s16
                return cc
            lax.fori_loop(0, EC2 // L, batch, 0)
            pltpu.sync_copy(con_v, contrib_h.at[pl.ds(eb, EC2)])
            return c
        lax.fori_loop(0, CH2, chunk1, 0)

        def ztbl(i, c):
            tbl_v[pl.ds(i * L, L)] = jnp.zeros((L,), jnp.float32)
            return c
        lax.fori_loop(0, N_MET_PAD // L, ztbl, 0)

        def chunk2(ci, c):
            eb = ebase + ci * EC2
            pltpu.sync_copy(met_h.at[pl.ds(eb, EC2)], idx_v)
            pltpu.sync_copy(contrib_h.at[pl.ds(eb, EC2)], con_v)

            def batch(b, cc):
                m16 = idx_v[pl.ds(b * L, L)]
                c16 = con_v[pl.ds(b * L, L)]
                plsc.addupdate_scatter(tbl_v, [m16], c16)
                return cc
            lax.fori_loop(0, EC2 // L, batch, 0)
            return c
        lax.fori_loop(0, CH2, chunk2, 0)

        pltpu.sync_copy(tbl_v, dacc_h.at[wid])

    return k34(v, rxn, sto, met)


def _k5_body(dacc_ref, conc_ref, p_ref, out_ref):
    lam = p_ref[0, 0]
    tgt = p_ref[0, 1]
    dsum = jnp.sum(dacc_ref[...], axis=0)[:N_MET]
    out_ref[0, :] = dsum - lam * (conc_ref[0] - tgt)


def _k5(dacc, conc, p):
    return pl.pallas_call(
        _k5_body,
        in_specs=[
            pl.BlockSpec((NW, N_MET_PAD), lambda: (0, 0)),
            pl.BlockSpec((1, N_MET), lambda: (0, 0)),
            pl.BlockSpec(memory_space=pltpu.SMEM),
        ],
        out_specs=pl.BlockSpec((1, N_MET), lambda: (0, 0)),
        out_shape=jax.ShapeDtypeStruct((1, N_MET), jnp.float32),
    )(dacc, conc, p)


def kernel(x, met_sub, rxn_sub, sto_sub, met_all, rxn_all, sto_all,
           W1, b1, W2, b2, V1, c1, V2, c2, log_k, p):
    conc = x[:, 3]
    met_s = met_sub.astype(jnp.int32)
    rxn_s = rxn_sub.astype(jnp.int32)
    met_a = met_all.astype(jnp.int32)
    rxn_a = rxn_all.astype(jnp.int32)

    ps = E_SUB_PAD - E_SUB
    met_s = jnp.pad(met_s, (0, ps))
    rxn_s = jnp.pad(rxn_s, (0, ps), constant_values=N_RXN)
    sto_s = jnp.pad(sto_sub, (0, ps))

    w1s = jnp.broadcast_to(W1[:, :, None], (HID, 2, L)) + 0.0
    b1s = jnp.broadcast_to(b1[:, None], (HID, L)) + 0.0
    w2s = jnp.broadcast_to(W2.T[:, :, None], (HID, MSG_DIM, L)) + 0.0
    b2s = jnp.broadcast_to(b2[:, None], (MSG_DIM, L)) + 0.0

    hacc = _k1(conc, met_s.reshape(-1, 128), rxn_s.reshape(-1, 128),
               sto_s, w1s, b1s, w2s, b2s)
    v = _k2(hacc[:, :N_RXN], V1, c1[None, :], V2.T + 0.0, c2,
            log_k[:, None])

    pa = E_ALL_PAD - E_ALL
    met_a = jnp.pad(met_a, (0, pa))
    rxn_a = jnp.pad(rxn_a, (0, pa))
    sto_a = jnp.pad(sto_all, (0, pa))

    dacc, _ = _k34(v[:, 0], rxn_a, sto_a, met_a)
    dxdt = _k5(dacc, conc[None, :], p)
    return dxdt.reshape(N_MET, 1)
